# baseline (speedup 1.0000x reference)
import functools

import jax
import jax.numpy as jnp
from jax import lax
from jax.experimental import pallas as pl
from jax.experimental.pallas import tpu as pltpu
from jax.experimental.pallas import tpu_sc as plsc

B, N_L, N_V, D = 16, 512, 576, 768
N_T = N_L + N_V
LANES = 16
NC, NS = 2, 16
HL = N_L // 2
HV = N_V // 2
CH = 72
NBUF = 2
KSL = D // LANES


def _add_rows(buf, trow, r0, r1):
    tvals = [trow[pl.ds(k * LANES, LANES)] for k in range(KSL)]

    def row_body(r, carry):
        for k in range(KSL):
            plsc.addupdate(buf.at[r, pl.ds(k * LANES, LANES)], tvals[k])
        return carry

    lax.fori_loop(r0, r1, row_body, 0, unroll=False)


def _fusion_body(vis_hbm, lang_hbm, mask_hbm, table_hbm,
                 out_hbm, omask_hbm,
                 buf0, buf1, trow_l, trow_v, mlbuf, mvbuf,
                 si0, si1, so0, so1):
    wid = lax.axis_index("s") * NC + lax.axis_index("c")
    b = wid // 2
    half = wid % 2
    bufs = (buf0, buf1)
    sin = (si0, si1)
    sout = (so0, so1)

    lang_sizes = (64, 64, 64, 64)
    vis_sizes = (72, 72, 72, 72)
    jobs = []
    r = half * HL
    for rows in lang_sizes:
        jobs.append((lang_hbm, r, r, trow_l, rows))
        r += rows
    r = half * HV
    for rows in vis_sizes:
        jobs.append((vis_hbm, r, N_L + r, trow_v, rows))
        r += rows
    NJ = len(jobs)

    def in_dma(c):
        src, srow, _, _, rows = jobs[c]
        return pltpu.make_async_copy(
            src.at[b, pl.ds(srow, rows), :],
            bufs[c % NBUF].at[pl.ds(0, rows), :], sin[c % NBUF])

    def out_dma(c):
        _, _, orow, _, rows = jobs[c]
        return pltpu.make_async_copy(
            bufs[c % NBUF].at[pl.ds(0, rows), :],
            out_hbm.at[b, pl.ds(orow, rows), :], sout[c % NBUF])

    pltpu.sync_copy(table_hbm.at[1], trow_l)
    pltpu.sync_copy(table_hbm.at[0], trow_v)
    in_dma(0).start()

    pltpu.sync_copy(mask_hbm.at[pl.ds(b * N_L + half * HL, HL)], mlbuf)
    pltpu.sync_copy(mlbuf, omask_hbm.at[pl.ds(b * N_T + half * HL, HL)])
    ones = jnp.ones((LANES,), jnp.int32)
    for k in range(HV // LANES):
        mvbuf[pl.ds(k * LANES, LANES)] = ones
    pltpu.sync_copy(mvbuf,
                    omask_hbm.at[pl.ds(b * N_T + N_L + half * HV, HV)])

    for c in range(NJ):
        in_dma(c).wait()
        if c >= 1:
            out_dma(c - 1).wait()
        if c + 1 < NJ:
            in_dma(c + 1).start()
        _add_rows(bufs[c % NBUF], jobs[c][3], 0, jobs[c][4])
        out_dma(c).start()

    out_dma(NJ - 1).wait()


@jax.jit
def _token_fusion(vision_tokens, language_tokens, language_mask, type_table):
    mesh = plsc.VectorSubcoreMesh(core_axis_name="c", subcore_axis_name="s")
    fn = functools.partial(
        pl.kernel,
        mesh=mesh,
        out_type=(
            jax.ShapeDtypeStruct((B, N_T, D), jnp.float32),
            jax.ShapeDtypeStruct((B * N_T,), jnp.int32),
        ),
        scratch_types=(
            [pltpu.VMEM((CH, D), jnp.float32)] * NBUF
            + [pltpu.VMEM((D,), jnp.float32)] * 2
            + [pltpu.VMEM((HL,), jnp.int32), pltpu.VMEM((HV,), jnp.int32)]
            + [pltpu.SemaphoreType.DMA] * (2 * NBUF)
        ),
    )(_fusion_body)
    fused, mask_flat = fn(vision_tokens, language_tokens,
                          language_mask.reshape(B * N_L), type_table)
    return fused, mask_flat.reshape(B, N_T)


def kernel(vision_tokens, language_tokens, language_mask, type_table):
    return _token_fusion(vision_tokens, language_tokens, language_mask,
                         type_table)
